# Initial kernel scaffold; baseline (speedup 1.0000x reference)
import functools
import jax
import jax.numpy as jnp
from jax import lax
from jax.experimental import pallas as pl
from jax.experimental.pallas import tpu as pltpu
from jax.experimental.pallas import tpu_sc as plsc

N = 10000
D = 128
NC, NS = 2, 16
NW = NC * NS
CHUNK = 128
K = 4
NPAD = 10240
ROWS_PER_TILE = NPAD // NS
BR = NPAD // 8
_SQRT2 = 1.4142135623730951


def _mesh():
    return plsc.VectorSubcoreMesh(
        core_axis_name="c", subcore_axis_name="s", num_cores=NC, num_subcores=NS
    )


def _sc_deg_body(dsti_hbm, ones_hbm, zeros_hbm, out_hbm, dsti_v, ones_v, ssem, acc_sh):
    cid = lax.axis_index("c")
    sid = lax.axis_index("s")
    wid = sid * NC + cid
    nch = dsti_v.shape[0]
    r0 = sid * ROWS_PER_TILE
    pltpu.sync_copy(zeros_hbm.at[pl.ds(r0, ROWS_PER_TILE)],
                    acc_sh.at[pl.ds(r0, ROWS_PER_TILE)])
    pltpu.sync_copy(dsti_hbm.at[wid], dsti_v)
    pltpu.sync_copy(ones_hbm, ones_v)
    plsc.subcore_barrier()

    kk = 8
    def group(g, carry):
        ds = [pltpu.async_copy(ones_v, acc_sh.at[dsti_v.at[g * kk + b]], ssem,
                               add=True)
              for b in range(kk)]
        for d in ds:
            d.wait()
        return carry
    lax.fori_loop(0, nch // kk, group, 0)

    plsc.subcore_barrier()
    pltpu.sync_copy(acc_sh.at[pl.ds(r0, ROWS_PER_TILE)],
                    out_hbm.at[cid, pl.ds(r0, ROWS_PER_TILE)])


def _sc_scatter_body(y_hbm, srci_hbm, dsti_hbm, zeros_hbm, out_hbm,
                     srci_v, dsti_v, buf, gsem, ssem, acc_sh):
    cid = lax.axis_index("c")
    sid = lax.axis_index("s")
    wid = sid * NC + cid
    nch = srci_v.shape[0]
    r0 = sid * ROWS_PER_TILE
    pltpu.sync_copy(zeros_hbm.at[pl.ds(r0, ROWS_PER_TILE)],
                    acc_sh.at[pl.ds(r0, ROWS_PER_TILE)])
    pltpu.sync_copy(srci_hbm.at[wid], srci_v)
    pltpu.sync_copy(dsti_hbm.at[wid], dsti_v)
    plsc.subcore_barrier()

    def group(g, carry):
        gds = [pltpu.async_copy(y_hbm.at[srci_v.at[g * K + b]], buf.at[b], gsem)
               for b in range(K)]
        for d in gds:
            d.wait()
        sds = [pltpu.async_copy(buf.at[b], acc_sh.at[dsti_v.at[g * K + b]], ssem,
                                add=True)
               for b in range(K)]
        for d in sds:
            d.wait()
        return carry
    lax.fori_loop(0, nch // K, group, 0)

    plsc.subcore_barrier()
    pltpu.sync_copy(acc_sh.at[pl.ds(r0, ROWS_PER_TILE)],
                    out_hbm.at[cid, pl.ds(r0, ROWS_PER_TILE)])


def _make_sc_calls(nch):
    deg_call = pl.kernel(
        _sc_deg_body,
        out_type=jax.ShapeDtypeStruct((NC, NPAD, 8), jnp.float32),
        mesh=_mesh(),
        scratch_types=[
            pltpu.VMEM((nch, CHUNK), jnp.int32),
            pltpu.VMEM((CHUNK, 8), jnp.float32),
            pltpu.SemaphoreType.DMA,
            pltpu.VMEM_SHARED((NPAD, 8), jnp.float32),
        ],
    )
    scat_call = pl.kernel(
        _sc_scatter_body,
        out_type=jax.ShapeDtypeStruct((NC, NPAD, D), jnp.float32),
        mesh=_mesh(),
        scratch_types=[
            pltpu.VMEM((nch, CHUNK), jnp.int32),
            pltpu.VMEM((nch, CHUNK), jnp.int32),
            pltpu.VMEM((K, CHUNK, D), jnp.float32),
            pltpu.SemaphoreType.DMA,
            pltpu.SemaphoreType.DMA,
            pltpu.VMEM_SHARED((NPAD, D), jnp.float32),
        ],
    )
    return deg_call, scat_call


def _dinv_block(degp_ref):
    deg = degp_ref[0, :, 0:1] + degp_ref[1, :, 0:1] + 1.0
    return lax.rsqrt(deg)


def _gelu(t):
    return t * 0.5 * (1.0 + lax.erf(t / _SQRT2))


def _tc_y1_body(x_ref, w_ref, degp_ref, y_ref):
    dinv = _dinv_block(degp_ref)
    xw = jnp.dot(x_ref[...], w_ref[...], preferred_element_type=jnp.float32)
    y_ref[...] = xw * dinv


def _layernorm(t, g_ref, b_ref):
    mu = jnp.mean(t, axis=-1, keepdims=True)
    var = jnp.mean((t - mu) ** 2, axis=-1, keepdims=True)
    return (t - mu) * lax.rsqrt(var + 1e-5) * g_ref[...] + b_ref[...]


def _tc_mid_body(accp_ref, y1_ref, degp_ref, b1_ref, g1_ref, be1_ref, w2_ref,
                 y2_ref):
    dinv = _dinv_block(degp_ref)
    t = (accp_ref[0] + accp_ref[1] + y1_ref[...]) * dinv + b1_ref[...]
    t = _layernorm(t, g1_ref, be1_ref)
    g = _gelu(t)
    y2_ref[...] = jnp.dot(g, w2_ref[...], preferred_element_type=jnp.float32) * dinv


def _tc_fin_body(accp_ref, y2_ref, degp_ref, b2_ref, g2_ref, be2_ref, x_ref,
                 out_ref):
    dinv = _dinv_block(degp_ref)
    t = (accp_ref[0] + accp_ref[1] + y2_ref[...]) * dinv + b2_ref[...]
    t = _layernorm(t, g2_ref, be2_ref)
    out_ref[...] = _gelu(t + x_ref[...])


def _row_spec():
    return pl.BlockSpec((BR, D), lambda i: (i, 0))


def _full_spec(shape):
    nd = len(shape)
    return pl.BlockSpec(shape, lambda i: (0,) * nd)


def _accp_spec():
    return pl.BlockSpec((2, BR, D), lambda i: (0, i, 0))


def _degp_spec():
    return pl.BlockSpec((2, BR, 8), lambda i: (0, i, 0))


def _vec_spec():
    return pl.BlockSpec((1, D), lambda i: (0, 0))


def _tc_y1(x_pad, W1, degp):
    return pl.pallas_call(
        _tc_y1_body,
        grid=(NPAD // BR,),
        in_specs=[_row_spec(), _full_spec((D, D)), _degp_spec()],
        out_specs=_row_spec(),
        out_shape=jax.ShapeDtypeStruct((NPAD, D), jnp.float32),
    )(x_pad, W1, degp)


def _tc_mid(accp, y1, degp, b1, g1, be1, W2):
    return pl.pallas_call(
        _tc_mid_body,
        grid=(NPAD // BR,),
        in_specs=[_accp_spec(), _row_spec(), _degp_spec(),
                  _vec_spec(), _vec_spec(), _vec_spec(), _full_spec((D, D))],
        out_specs=_row_spec(),
        out_shape=jax.ShapeDtypeStruct((NPAD, D), jnp.float32),
    )(accp, y1, degp, b1, g1, be1, W2)


def _tc_fin(accp, y2, degp, b2, g2, be2, x_pad):
    return pl.pallas_call(
        _tc_fin_body,
        grid=(NPAD // BR,),
        in_specs=[_accp_spec(), _row_spec(), _degp_spec(),
                  _vec_spec(), _vec_spec(), _vec_spec(), _row_spec()],
        out_specs=_row_spec(),
        out_shape=jax.ShapeDtypeStruct((NPAD, D), jnp.float32),
    )(accp, y2, degp, b2, g2, be2, x_pad)


@jax.jit
def kernel(x, edge_index, batch, W1, b1, gamma1, beta1, W2, b2, gamma2, beta2):
    E = edge_index.shape[1]
    epad = ((E + NW * CHUNK - 1) // (NW * CHUNK)) * NW * CHUNK
    nch = epad // (NW * CHUNK)

    src = edge_index[0].astype(jnp.int32)
    dst = edge_index[1].astype(jnp.int32)
    pad = N + (jnp.arange(epad - E, dtype=jnp.int32) % (NPAD - N))
    srci = jnp.concatenate([src, pad]).reshape(NW, nch, CHUNK)
    dsti = jnp.concatenate([dst, pad]).reshape(NW, nch, CHUNK)

    x_pad = jnp.pad(x, ((0, NPAD - N), (0, 0)))
    zeros128 = jnp.zeros((NPAD, D), jnp.float32)
    zeros8 = jnp.zeros((NPAD, 8), jnp.float32)
    ones8 = jnp.ones((CHUNK, 8), jnp.float32)

    deg_call, scat_call = _make_sc_calls(nch)

    degp = deg_call(dsti, ones8, zeros8)
    y1 = _tc_y1(x_pad, W1, degp)
    acc1 = scat_call(y1, srci, dsti, zeros128)
    y2 = _tc_mid(acc1, y1, degp, b1.reshape(1, D), gamma1.reshape(1, D),
                 beta1.reshape(1, D), W2)
    acc2 = scat_call(y2, srci, dsti, zeros128)
    out = _tc_fin(acc2, y2, degp, b2.reshape(1, D), gamma2.reshape(1, D),
                  beta2.reshape(1, D), x_pad)
    return out[:N]


# speedup vs baseline: 4.5668x; 4.5668x over previous
import jax
import jax.numpy as jnp
from jax import lax
from jax.experimental import pallas as pl
from jax.experimental.pallas import tpu as pltpu
from jax.experimental.pallas import tpu_sc as plsc

N = 10000
D = 128
NC, NS = 2, 16
NW = NC * NS
CHUNK = 128
NSUB = 8
NPAD = 10240
ROWS_PER_TILE = NPAD // NS
RSUB = NPAD * NSUB // NS
BR = NPAD // 8
_SQRT2 = 1.4142135623730951


def _mesh():
    return plsc.VectorSubcoreMesh(
        core_axis_name="c", subcore_axis_name="s", num_cores=NC, num_subcores=NS
    )


def _sc_deg_body(dsti_hbm, ones_hbm, zeros_hbm, out_hbm, dsti_v, ones_v, ssem,
                 acc_sh):
    cid = lax.axis_index("c")
    sid = lax.axis_index("s")
    wid = sid * NC + cid
    nch = dsti_v.shape[0]
    r0 = sid * ROWS_PER_TILE
    pltpu.sync_copy(zeros_hbm.at[pl.ds(r0, ROWS_PER_TILE)],
                    acc_sh.at[pl.ds(r0, ROWS_PER_TILE)])
    pltpu.sync_copy(dsti_hbm.at[wid], dsti_v)
    pltpu.sync_copy(ones_hbm, ones_v)
    plsc.subcore_barrier()

    kk = 8
    def group(g, carry):
        ds = [pltpu.async_copy(ones_v, acc_sh.at[dsti_v.at[g * kk + b]], ssem,
                               add=True)
              for b in range(kk)]
        for d in ds:
            d.wait()
        return carry
    lax.fori_loop(0, nch // kk, group, 0)

    plsc.subcore_barrier()
    pltpu.sync_copy(acc_sh.at[pl.ds(r0, ROWS_PER_TILE)],
                    out_hbm.at[cid, pl.ds(r0, ROWS_PER_TILE)])


def _sc_gather_body(y_hbm, srci_hbm, out_hbm, srci_v, buf, gsem):
    cid = lax.axis_index("c")
    sid = lax.axis_index("s")
    wid = sid * NC + cid
    nch = srci_v.shape[0]
    pltpu.sync_copy(srci_hbm.at[wid], srci_v)

    def chunk(j, carry):
        pltpu.async_copy(y_hbm.at[srci_v.at[j]], buf, gsem).wait()
        pltpu.sync_copy(buf, out_hbm.at[wid, j])
        return carry
    lax.fori_loop(0, nch, chunk, 0)


def _sc_scatter_body(g3_hbm, dsti_hbm, zeros_hbm, out_hbm,
                     dsti_v, bufa, ssema, a0, a1, a2, a3, a4, a5, a6, a7):
    cid = lax.axis_index("c")
    sid = lax.axis_index("s")
    wid = sid * NC + cid
    nch = dsti_v.shape[0]
    accs = (a0, a1, a2, a3, a4, a5, a6, a7)
    r0 = sid * ROWS_PER_TILE
    for a in accs:
        pltpu.sync_copy(zeros_hbm.at[pl.ds(r0, ROWS_PER_TILE)],
                        a.at[pl.ds(r0, ROWS_PER_TILE)])
    pltpu.sync_copy(dsti_hbm.at[wid], dsti_v)
    plsc.subcore_barrier()

    def chunk(j, carry):
        for t in range(NSUB):
            pltpu.sync_copy(g3_hbm.at[wid, j * NSUB + t], bufa)
            pltpu.async_copy(bufa, accs[t].at[dsti_v.at[j]], ssema,
                             add=True).wait()
        return carry
    lax.fori_loop(0, nch, chunk, 0)

    plsc.subcore_barrier()
    for t in range(NSUB):
        pltpu.sync_copy(accs[t].at[pl.ds(r0, ROWS_PER_TILE)],
                        out_hbm.at[cid, t, pl.ds(r0, ROWS_PER_TILE)])


def _make_sc_calls(nch):
    deg_call = pl.kernel(
        _sc_deg_body,
        out_type=jax.ShapeDtypeStruct((NC, NPAD, 16), jnp.float32),
        mesh=_mesh(),
        scratch_types=[
            pltpu.VMEM((nch, CHUNK), jnp.int32),
            pltpu.VMEM((CHUNK, 16), jnp.float32),
            pltpu.SemaphoreType.DMA,
            pltpu.VMEM_SHARED((NPAD, 16), jnp.float32),
        ],
    )
    gath_call = pl.kernel(
        _sc_gather_body,
        out_type=jax.ShapeDtypeStruct((NW, nch, CHUNK, D), jnp.float32),
        mesh=_mesh(),
        scratch_types=[
            pltpu.VMEM((nch, CHUNK), jnp.int32),
            pltpu.VMEM((CHUNK, D), jnp.float32),
            pltpu.SemaphoreType.DMA,
        ],
    )
    scat_call = pl.kernel(
        _sc_scatter_body,
        out_type=jax.ShapeDtypeStruct((NC, NSUB, NPAD, 16), jnp.float32),
        mesh=_mesh(),
        scratch_types=[
            pltpu.VMEM((nch, CHUNK), jnp.int32),
            pltpu.VMEM((CHUNK, 16), jnp.float32),
            pltpu.SemaphoreType.DMA,
        ] + [pltpu.VMEM_SHARED((NPAD, 16), jnp.float32)] * 8,
    )
    return deg_call, gath_call, scat_call


def _dinv_block(degp_ref):
    deg = degp_ref[0, :, 0:1] + degp_ref[1, :, 0:1] + 1.0
    return lax.rsqrt(deg)


def _gelu(t):
    return t * 0.5 * (1.0 + lax.erf(t / _SQRT2))


def _layernorm(t, g_ref, b_ref):
    mu = jnp.mean(t, axis=-1, keepdims=True)
    var = jnp.mean((t - mu) ** 2, axis=-1, keepdims=True)
    return (t - mu) * lax.rsqrt(var + 1e-5) * g_ref[...] + b_ref[...]


def _tc_y1_body(x_ref, w_ref, degp_ref, y_ref):
    dinv = _dinv_block(degp_ref)
    xw = jnp.dot(x_ref[...], w_ref[...], preferred_element_type=jnp.float32)
    y_ref[...] = xw * dinv


def _tc_mid_body(accp_ref, y1_ref, degp_ref, b1_ref, g1_ref, be1_ref, w2_ref,
                 y2_ref):
    dinv = _dinv_block(degp_ref)
    t = (accp_ref[0] + accp_ref[1] + y1_ref[...]) * dinv + b1_ref[...]
    t = _layernorm(t, g1_ref, be1_ref)
    g = _gelu(t)
    y2_ref[...] = jnp.dot(g, w2_ref[...], preferred_element_type=jnp.float32) * dinv


def _tc_fin_body(accp_ref, y2_ref, degp_ref, b2_ref, g2_ref, be2_ref, x_ref,
                 out_ref):
    dinv = _dinv_block(degp_ref)
    t = (accp_ref[0] + accp_ref[1] + y2_ref[...]) * dinv + b2_ref[...]
    t = _layernorm(t, g2_ref, be2_ref)
    out_ref[...] = _gelu(t + x_ref[...])


def _row_spec():
    return pl.BlockSpec((BR, D), lambda i: (i, 0))


def _full_spec(shape):
    nd = len(shape)
    return pl.BlockSpec(shape, lambda i: (0,) * nd)


def _accp_spec():
    return pl.BlockSpec((2, BR, D), lambda i: (0, i, 0))


def _degp_spec():
    return pl.BlockSpec((2, BR, 16), lambda i: (0, i, 0))


def _vec_spec():
    return pl.BlockSpec((1, D), lambda i: (0, 0))


def _tc_y1(x_pad, W1, degp):
    return pl.pallas_call(
        _tc_y1_body,
        grid=(NPAD // BR,),
        in_specs=[_row_spec(), _full_spec((D, D)), _degp_spec()],
        out_specs=_row_spec(),
        out_shape=jax.ShapeDtypeStruct((NPAD, D), jnp.float32),
    )(x_pad, W1, degp)


def _tc_mid(accp, y1, degp, b1, g1, be1, W2):
    return pl.pallas_call(
        _tc_mid_body,
        grid=(NPAD // BR,),
        in_specs=[_accp_spec(), _row_spec(), _degp_spec(),
                  _vec_spec(), _vec_spec(), _vec_spec(), _full_spec((D, D))],
        out_specs=_row_spec(),
        out_shape=jax.ShapeDtypeStruct((NPAD, D), jnp.float32),
    )(accp, y1, degp, b1, g1, be1, W2)


def _tc_fin(accp, y2, degp, b2, g2, be2, x_pad):
    return pl.pallas_call(
        _tc_fin_body,
        grid=(NPAD // BR,),
        in_specs=[_accp_spec(), _row_spec(), _degp_spec(),
                  _vec_spec(), _vec_spec(), _vec_spec(), _row_spec()],
        out_specs=_row_spec(),
        out_shape=jax.ShapeDtypeStruct((NPAD, D), jnp.float32),
    )(accp, y2, degp, b2, g2, be2, x_pad)


@jax.jit
def kernel(x, edge_index, batch, W1, b1, gamma1, beta1, W2, b2, gamma2, beta2):
    E = edge_index.shape[1]
    nch = ((E + NW * CHUNK - 1) // (NW * CHUNK) + 7) // 8 * 8
    epad = nch * NW * CHUNK

    src = edge_index[0].astype(jnp.int32)
    dst = edge_index[1].astype(jnp.int32)
    fill = N + (jnp.arange(epad - E, dtype=jnp.int32) % (NPAD - N))
    src_p = jnp.concatenate([src, fill])
    dst_p = jnp.concatenate([dst, fill])
    srci = src_p.reshape(NW, nch, CHUNK)
    dsti = dst_p.reshape(NW, nch, CHUNK)
    x_pad = jnp.pad(x, ((0, NPAD - N), (0, 0)))
    zeros16w = jnp.zeros((NPAD, 16), jnp.float32)
    ones16w = jnp.ones((CHUNK, 16), jnp.float32)

    deg_call, gath_call, scat_call = _make_sc_calls(nch)

    dst_flat = dst_p

    def edge_agg(y):
        g = gath_call(y, srci).reshape(epad, D)
        acc = jnp.zeros((NPAD, D), jnp.float32).at[dst_flat].add(g)
        return jnp.stack([acc, jnp.zeros((NPAD, D), jnp.float32)])

    degp = deg_call(dsti, ones16w, zeros16w)
    y1 = _tc_y1(x_pad, W1, degp)
    acc1 = edge_agg(y1)
    y2 = _tc_mid(acc1, y1, degp, b1.reshape(1, D), gamma1.reshape(1, D),
                 beta1.reshape(1, D), W2)
    acc2 = edge_agg(y2)
    out = _tc_fin(acc2, y2, degp, b2.reshape(1, D), gamma2.reshape(1, D),
                  beta2.reshape(1, D), x_pad)
    return out[:N]
